# Initial kernel scaffold; baseline (speedup 1.0000x reference)
import jax
import jax.numpy as jnp
from jax.experimental import pallas as pl


def kernel(x, edge_index, W1, b1, W2, b2):
    raise NotImplementedError("write your pallas kernel here")


# speedup vs baseline: 5.5279x; 5.5279x over previous
import functools

import jax
import jax.numpy as jnp
from jax import lax
from jax.experimental import pallas as pl
from jax.experimental.pallas import tpu as pltpu
from jax.experimental.pallas import tpu_sc as plsc

N_NODES = 10000
NP = 10240
PAD_NODE = NP - 1
D = 128
CHUNK = 128
NROWS = 2560
EP = NROWS * CHUNK
HALF = NP // 2
JUNK = HALF
ACC_R = HALF + 8
CPT_H = NROWS // 32
CPT_A = NROWS // 16
HIST_RPT = NP // 16
AGG_RPT = HALF // 16

_mesh = plsc.VectorSubcoreMesh(core_axis_name="c", subcore_axis_name="s")


@functools.partial(
    pl.kernel,
    out_type=jax.ShapeDtypeStruct((2 * NP,), jnp.float32),
    mesh=_mesh,
    scratch_types=[
        pltpu.VMEM((CPT_H, CHUNK), jnp.int32),
        pltpu.VMEM((CPT_H, CHUNK), jnp.int32),
        pltpu.VMEM((CHUNK,), jnp.float32),
        pltpu.VMEM((HIST_RPT,), jnp.float32),
        pltpu.VMEM_SHARED((NP,), jnp.float32),
    ],
)
def _deg_hist(edges_hbm, out_hbm, pk_v, dst_v, ones_v, stage_v, hist):
    cid = lax.axis_index("c")
    sid = lax.axis_index("s")
    wid = sid * 2 + cid
    pltpu.sync_copy(edges_hbm.at[pl.ds(wid * CPT_H, CPT_H)], pk_v)

    mask14 = jnp.full((16,), 16383, jnp.int32)

    def unpack_body(i, _):
        r = i // 8
        s = (i % 8) * 16
        dst_v[r, pl.ds(s, 16)] = pk_v[r, pl.ds(s, 16)] & mask14
        return _

    lax.fori_loop(0, CPT_H * (CHUNK // 16), unpack_body, 0)

    ones16 = jnp.ones((16,), jnp.float32)
    for j in range(CHUNK // 16):
        ones_v[pl.ds(j * 16, 16)] = ones16

    base = sid * HIST_RPT
    zeros16 = jnp.zeros((16,), jnp.float32)

    def zstage_body(j, _):
        stage_v[pl.ds(j * 16, 16)] = zeros16
        return _

    lax.fori_loop(0, HIST_RPT // 16, zstage_body, 0)
    pltpu.sync_copy(stage_v, hist.at[pl.ds(base, HIST_RPT)])
    plsc.subcore_barrier()

    def hist_body(c, _):
        pltpu.sync_copy(ones_v, hist.at[dst_v.at[c]], add=True)
        return _

    lax.fori_loop(0, CPT_H, hist_body, 0)
    plsc.subcore_barrier()

    pltpu.sync_copy(hist.at[pl.ds(base, HIST_RPT)], stage_v)
    pltpu.sync_copy(stage_v, out_hbm.at[pl.ds(cid * NP + base, HIST_RPT)])


@functools.partial(
    pl.kernel,
    out_type=jax.ShapeDtypeStruct((NP, D), jnp.float32),
    mesh=_mesh,
    scratch_types=[
        pltpu.VMEM((CPT_A, CHUNK), jnp.int32),
        pltpu.VMEM((CPT_A, CHUNK), jnp.int32),
        pltpu.VMEM((CPT_A, CHUNK), jnp.int32),
        pltpu.VMEM((CHUNK, D), jnp.float32),
        pltpu.VMEM((64, D), jnp.float32),
        pltpu.VMEM_SHARED((ACC_R, D), jnp.float32),
        pltpu.SemaphoreType.DMA,
    ],
)
def _edge_agg(g_hbm, edges_hbm, out_hbm,
              pk_v, src_v, dst_v, rows_v, stage_v, acc, gsem):
    cid = lax.axis_index("c")
    sid = lax.axis_index("s")

    pltpu.sync_copy(edges_hbm.at[pl.ds(sid * CPT_A, CPT_A)], pk_v)

    mask14 = jnp.full((16,), 16383, jnp.int32)
    sh14 = jnp.full((16,), 14, jnp.int32)
    lo = cid * HALF
    junk16 = jnp.full((16,), JUNK, jnp.int32)

    def unpack_body(i, _):
        r = i // 8
        s = (i % 8) * 16
        p = pk_v[r, pl.ds(s, 16)]
        src_v[r, pl.ds(s, 16)] = lax.shift_right_logical(p, sh14)
        dl = (p & mask14) - lo
        ok = (dl >= 0) & (dl < HALF)
        dst_v[r, pl.ds(s, 16)] = jnp.where(ok, dl, junk16)
        return _

    lax.fori_loop(0, CPT_A * (CHUNK // 16), unpack_body, 0)

    zeros16 = jnp.zeros((16,), jnp.float32)

    def zrow_body(r, _):
        for j in range(D // 16):
            stage_v[r, pl.ds(j * 16, 16)] = zeros16
        return _

    lax.fori_loop(0, 64, zrow_body, 0)
    base = sid * AGG_RPT
    for k in range(AGG_RPT // 64):
        pltpu.sync_copy(stage_v, acc.at[pl.ds(base + k * 64, 64)])
    plsc.subcore_barrier()

    def chunk_body(c, _):
        pltpu.async_copy(g_hbm.at[src_v.at[c]], rows_v, gsem).wait()
        pltpu.sync_copy(rows_v, acc.at[dst_v.at[c]], add=True)
        return _

    lax.fori_loop(0, CPT_A, chunk_body, 0)
    plsc.subcore_barrier()

    for k in range(AGG_RPT // 64):
        r = base + k * 64
        pltpu.sync_copy(acc.at[pl.ds(r, 64)], stage_v)
        pltpu.sync_copy(stage_v, out_hbm.at[pl.ds(cid * HALF + r, 64)])


_BR = 256
_NB = NP // _BR


def _tc_first_body(deg_ref, x_ref, w_ref, g_ref, dis_ref):
    deg = deg_ref[0, :] + deg_ref[1, :] + 1.0
    dis = lax.rsqrt(deg)
    h = jnp.dot(x_ref[...], w_ref[...], preferred_element_type=jnp.float32)
    g_ref[...] = h * dis[:, None]
    dis_ref[...] = dis[:, None]


def _tc_first(deg_part, x_p, w1):
    return pl.pallas_call(
        _tc_first_body,
        grid=(_NB,),
        in_specs=[
            pl.BlockSpec((2, _BR), lambda i: (0, i)),
            pl.BlockSpec((_BR, D), lambda i: (i, 0)),
            pl.BlockSpec((D, D), lambda i: (0, 0)),
        ],
        out_specs=[
            pl.BlockSpec((_BR, D), lambda i: (i, 0)),
            pl.BlockSpec((_BR, 1), lambda i: (i, 0)),
        ],
        out_shape=[
            jax.ShapeDtypeStruct((NP, D), jnp.float32),
            jax.ShapeDtypeStruct((NP, 1), jnp.float32),
        ],
    )(deg_part, x_p, w1)


def _tc_mid_body(s_ref, g_ref, dis_ref, b_ref, w_ref, out_ref):
    dis = dis_ref[...]
    y = dis * (s_ref[...] + g_ref[...]) + b_ref[...]
    y = jnp.maximum(y, 0.0)
    h = jnp.dot(y, w_ref[...], preferred_element_type=jnp.float32)
    out_ref[...] = h * dis


def _tc_mid(s, g1, dis, b1, w2):
    return pl.pallas_call(
        _tc_mid_body,
        grid=(_NB,),
        in_specs=[
            pl.BlockSpec((_BR, D), lambda i: (i, 0)),
            pl.BlockSpec((_BR, D), lambda i: (i, 0)),
            pl.BlockSpec((_BR, 1), lambda i: (i, 0)),
            pl.BlockSpec((1, D), lambda i: (0, 0)),
            pl.BlockSpec((D, D), lambda i: (0, 0)),
        ],
        out_specs=pl.BlockSpec((_BR, D), lambda i: (i, 0)),
        out_shape=jax.ShapeDtypeStruct((NP, D), jnp.float32),
    )(s, g1, dis, b1, w2)


def _tc_out_body(s_ref, g_ref, dis_ref, b_ref, out_ref):
    out_ref[...] = dis_ref[...] * (s_ref[...] + g_ref[...]) + b_ref[...]


def _tc_out(s, g2, dis, b2):
    return pl.pallas_call(
        _tc_out_body,
        grid=(_NB,),
        in_specs=[
            pl.BlockSpec((_BR, D), lambda i: (i, 0)),
            pl.BlockSpec((_BR, D), lambda i: (i, 0)),
            pl.BlockSpec((_BR, 1), lambda i: (i, 0)),
            pl.BlockSpec((1, D), lambda i: (0, 0)),
        ],
        out_specs=pl.BlockSpec((_BR, D), lambda i: (i, 0)),
        out_shape=jax.ShapeDtypeStruct((NP, D), jnp.float32),
    )(s, g2, dis, b2)


def kernel(x, edge_index, W1, b1, W2, b2):
    E = edge_index.shape[1]
    packed = (edge_index[0] << 14) | edge_index[1]
    pad = jnp.full((EP - E,), (PAD_NODE << 14) | PAD_NODE, jnp.int32)
    packed2d = jnp.concatenate([packed, pad]).reshape(NROWS, CHUNK)

    x_p = jnp.zeros((NP, D), x.dtype).at[:N_NODES].set(x)
    b1r = b1.reshape(1, D)
    b2r = b2.reshape(1, D)

    deg_part = _deg_hist(packed2d).reshape(2, NP)
    g1, dis = _tc_first(deg_part, x_p, W1)

    s1 = _edge_agg(g1, packed2d)
    g2 = _tc_mid(s1, g1, dis, b1r, W2)

    s2 = _edge_agg(g2, packed2d)
    out = _tc_out(s2, g2, dis, b2r)
    return out[:N_NODES]


# speedup vs baseline: 5.8416x; 1.0567x over previous
import functools

import jax
import jax.numpy as jnp
from jax import lax
from jax.experimental import pallas as pl
from jax.experimental.pallas import tpu as pltpu
from jax.experimental.pallas import tpu_sc as plsc

N_NODES = 10000
NP = 10240
PAD_NODE = NP - 1
D = 128
CHUNK = 128
NROWS = 2560
EP = NROWS * CHUNK
HALF = NP // 2
JUNK = HALF
ACC_R = HALF + 8
CPT_H = NROWS // 32
CPT_A = NROWS // 16
HIST_RPT = NP // 16
AGG_RPT = HALF // 16
GCH = 128
NCH = CPT_A * CHUNK // GCH

_mesh = plsc.VectorSubcoreMesh(core_axis_name="c", subcore_axis_name="s")


@functools.partial(
    pl.kernel,
    out_type=jax.ShapeDtypeStruct((2 * NP,), jnp.float32),
    mesh=_mesh,
    scratch_types=[
        pltpu.VMEM((CPT_H, CHUNK), jnp.int32),
        pltpu.VMEM((CPT_H, CHUNK), jnp.int32),
        pltpu.VMEM((CHUNK,), jnp.float32),
        pltpu.VMEM((HIST_RPT,), jnp.float32),
        pltpu.VMEM_SHARED((NP,), jnp.float32),
    ],
)
def _deg_hist(edges_hbm, out_hbm, pk_v, dst_v, ones_v, stage_v, hist):
    cid = lax.axis_index("c")
    sid = lax.axis_index("s")
    wid = sid * 2 + cid
    pltpu.sync_copy(edges_hbm.at[pl.ds(wid * CPT_H, CPT_H)], pk_v)

    mask14 = jnp.full((16,), 16383, jnp.int32)

    def unpack_body(i, _):
        r = i // 8
        s = (i % 8) * 16
        dst_v[r, pl.ds(s, 16)] = pk_v[r, pl.ds(s, 16)] & mask14
        return _

    lax.fori_loop(0, CPT_H * (CHUNK // 16), unpack_body, 0)

    ones16 = jnp.ones((16,), jnp.float32)
    for j in range(CHUNK // 16):
        ones_v[pl.ds(j * 16, 16)] = ones16

    base = sid * HIST_RPT
    zeros16 = jnp.zeros((16,), jnp.float32)

    def zstage_body(j, _):
        stage_v[pl.ds(j * 16, 16)] = zeros16
        return _

    lax.fori_loop(0, HIST_RPT // 16, zstage_body, 0)
    pltpu.sync_copy(stage_v, hist.at[pl.ds(base, HIST_RPT)])
    plsc.subcore_barrier()

    def hist_body(c, _):
        pltpu.sync_copy(ones_v, hist.at[dst_v.at[c]], add=True)
        return _

    lax.fori_loop(0, CPT_H, hist_body, 0)
    plsc.subcore_barrier()

    pltpu.sync_copy(hist.at[pl.ds(base, HIST_RPT)], stage_v)
    pltpu.sync_copy(stage_v, out_hbm.at[pl.ds(cid * NP + base, HIST_RPT)])


@functools.partial(
    pl.kernel,
    out_type=jax.ShapeDtypeStruct((NP, D), jnp.float32),
    mesh=_mesh,
    scratch_types=[
        pltpu.VMEM((NCH, GCH), jnp.int32),
        pltpu.VMEM((NCH, GCH), jnp.int32),
        pltpu.VMEM((GCH, D), jnp.float32),
        pltpu.VMEM((GCH, D), jnp.float32),
        pltpu.VMEM((64, D), jnp.float32),
        pltpu.VMEM_SHARED((ACC_R, D), jnp.float32),
        pltpu.SemaphoreType.DMA,
    ],
)
def _edge_agg(g_hbm, edges_hbm, out_hbm,
              src_v, dst_v, rows_a, rows_b, stage_v, acc, gsem):
    cid = lax.axis_index("c")
    sid = lax.axis_index("s")

    pltpu.sync_copy(edges_hbm.at[pl.ds(sid * CPT_A, CPT_A)], src_v)

    mask14 = jnp.full((16,), 16383, jnp.int32)
    sh14 = jnp.full((16,), 14, jnp.int32)
    lo = cid * HALF
    junk16 = jnp.full((16,), JUNK, jnp.int32)

    def unpack_body(i, _):
        r = i // 8
        s = (i % 8) * 16
        p = src_v[r, pl.ds(s, 16)]
        src_v[r, pl.ds(s, 16)] = lax.shift_right_logical(p, sh14)
        dl = (p & mask14) - lo
        ok = (dl >= 0) & (dl < HALF)
        dst_v[r, pl.ds(s, 16)] = jnp.where(ok, dl, junk16)
        return _

    lax.fori_loop(0, CPT_A * (CHUNK // 16), unpack_body, 0)

    zeros16 = jnp.zeros((16,), jnp.float32)

    def zrow_body(r, _):
        for j in range(D // 16):
            stage_v[r, pl.ds(j * 16, 16)] = zeros16
        return _

    lax.fori_loop(0, 64, zrow_body, 0)
    base = sid * AGG_RPT
    for k in range(AGG_RPT // 64):
        pltpu.sync_copy(stage_v, acc.at[pl.ds(base + k * 64, 64)])
    plsc.subcore_barrier()

    pltpu.async_copy(g_hbm.at[src_v.at[0]], rows_a, gsem)

    def chunk_body(k, carry):
        c0 = 2 * k
        pltpu.async_copy(g_hbm.at[src_v.at[c0 + 1]], rows_b, gsem)
        pltpu.make_async_copy(g_hbm.at[src_v.at[c0]], rows_a, gsem).wait()
        pltpu.sync_copy(rows_a, acc.at[dst_v.at[c0]], add=True)

        @pl.when(c0 + 2 < NCH)
        def _next():
            pltpu.async_copy(g_hbm.at[src_v.at[c0 + 2]], rows_a, gsem)

        pltpu.make_async_copy(g_hbm.at[src_v.at[c0 + 1]], rows_b, gsem).wait()
        pltpu.sync_copy(rows_b, acc.at[dst_v.at[c0 + 1]], add=True)
        return carry

    lax.fori_loop(0, NCH // 2, chunk_body, 0)
    plsc.subcore_barrier()

    for k in range(AGG_RPT // 64):
        r = base + k * 64
        pltpu.sync_copy(acc.at[pl.ds(r, 64)], stage_v)
        pltpu.sync_copy(stage_v, out_hbm.at[pl.ds(cid * HALF + r, 64)])


_BR = 256
_NB = NP // _BR


def _tc_first_body(deg_ref, x_ref, w_ref, g_ref, dis_ref):
    deg = deg_ref[0, :] + deg_ref[1, :] + 1.0
    dis = lax.rsqrt(deg)
    h = jnp.dot(x_ref[...], w_ref[...], preferred_element_type=jnp.float32)
    g_ref[...] = h * dis[:, None]
    dis_ref[...] = dis[:, None]


def _tc_first(deg_part, x_p, w1):
    return pl.pallas_call(
        _tc_first_body,
        grid=(_NB,),
        in_specs=[
            pl.BlockSpec((2, _BR), lambda i: (0, i)),
            pl.BlockSpec((_BR, D), lambda i: (i, 0)),
            pl.BlockSpec((D, D), lambda i: (0, 0)),
        ],
        out_specs=[
            pl.BlockSpec((_BR, D), lambda i: (i, 0)),
            pl.BlockSpec((_BR, 1), lambda i: (i, 0)),
        ],
        out_shape=[
            jax.ShapeDtypeStruct((NP, D), jnp.float32),
            jax.ShapeDtypeStruct((NP, 1), jnp.float32),
        ],
    )(deg_part, x_p, w1)


def _tc_mid_body(s_ref, g_ref, dis_ref, b_ref, w_ref, out_ref):
    dis = dis_ref[...]
    y = dis * (s_ref[...] + g_ref[...]) + b_ref[...]
    y = jnp.maximum(y, 0.0)
    h = jnp.dot(y, w_ref[...], preferred_element_type=jnp.float32)
    out_ref[...] = h * dis


def _tc_mid(s, g1, dis, b1, w2):
    return pl.pallas_call(
        _tc_mid_body,
        grid=(_NB,),
        in_specs=[
            pl.BlockSpec((_BR, D), lambda i: (i, 0)),
            pl.BlockSpec((_BR, D), lambda i: (i, 0)),
            pl.BlockSpec((_BR, 1), lambda i: (i, 0)),
            pl.BlockSpec((1, D), lambda i: (0, 0)),
            pl.BlockSpec((D, D), lambda i: (0, 0)),
        ],
        out_specs=pl.BlockSpec((_BR, D), lambda i: (i, 0)),
        out_shape=jax.ShapeDtypeStruct((NP, D), jnp.float32),
    )(s, g1, dis, b1, w2)


def _tc_out_body(s_ref, g_ref, dis_ref, b_ref, out_ref):
    out_ref[...] = dis_ref[...] * (s_ref[...] + g_ref[...]) + b_ref[...]


def _tc_out(s, g2, dis, b2):
    return pl.pallas_call(
        _tc_out_body,
        grid=(_NB,),
        in_specs=[
            pl.BlockSpec((_BR, D), lambda i: (i, 0)),
            pl.BlockSpec((_BR, D), lambda i: (i, 0)),
            pl.BlockSpec((_BR, 1), lambda i: (i, 0)),
            pl.BlockSpec((1, D), lambda i: (0, 0)),
        ],
        out_specs=pl.BlockSpec((_BR, D), lambda i: (i, 0)),
        out_shape=jax.ShapeDtypeStruct((NP, D), jnp.float32),
    )(s, g2, dis, b2)


def kernel(x, edge_index, W1, b1, W2, b2):
    E = edge_index.shape[1]
    packed = (edge_index[0] << 14) | edge_index[1]
    pad = jnp.full((EP - E,), (PAD_NODE << 14) | PAD_NODE, jnp.int32)
    packed2d = jnp.concatenate([packed, pad]).reshape(NROWS, CHUNK)

    x_p = jnp.zeros((NP, D), x.dtype).at[:N_NODES].set(x)
    b1r = b1.reshape(1, D)
    b2r = b2.reshape(1, D)

    deg_part = _deg_hist(packed2d).reshape(2, NP)
    g1, dis = _tc_first(deg_part, x_p, W1)

    s1 = _edge_agg(g1, packed2d)
    g2 = _tc_mid(s1, g1, dis, b1r, W2)

    s2 = _edge_agg(g2, packed2d)
    out = _tc_out(s2, g2, dis, b2r)
    return out[:N_NODES]
